# baseline (speedup 1.0000x reference)
import functools

import jax
import jax.numpy as jnp
from jax import lax
from jax.experimental import pallas as pl
from jax.experimental.pallas import tpu as pltpu
from jax.experimental.pallas import tpu_sc as plsc

_B = 16384
_VOCAB = 5
_EMB = 4
_NF = 3
_OD = 8
_NC = 2
_NS = 16
_NW = _NC * _NS
_RPT = _B // _NW
_TPAD = 48
_CHUNKS = 4
_RPC = _RPT // _CHUNKS


def _body(ids_hbm, num_hbm, emb_hbm, w_hbm, b_hbm, out_hbm,
          ids_v, num_v, out_v, emb_v, w_v, b_v, t_v,
          sem_big, sem_small, sem_out):
    wid = lax.axis_index("s") * _NC + lax.axis_index("c")
    base = wid * _RPT

    cp_ids = pltpu.async_copy(ids_hbm.at[pl.ds(base, _RPT)], ids_v, sem_big)
    cp_num = pltpu.async_copy(num_hbm.at[pl.ds(base, _RPT)], num_v, sem_big)
    cp_emb = pltpu.async_copy(emb_hbm, emb_v, sem_small)
    cp_w = pltpu.async_copy(w_hbm, w_v, sem_small)
    cp_b = pltpu.async_copy(b_hbm, b_v, sem_small)
    cp_emb.wait()
    cp_w.wait()
    cp_b.wait()

    iota = lax.iota(jnp.int32, 16)

    for g in range(_TPAD // 16):
        e = iota + g * 16
        v_idx = jnp.minimum(e >> 3, _VOCAB - 1)
        j_idx = e & 7
        tv = plsc.load_gather(b_v, [j_idx])
        for k in range(_EMB):
            kf = jnp.full((16,), k, jnp.int32)
            tv = tv + (plsc.load_gather(emb_v, [v_idx, kf]) *
                       plsc.load_gather(w_v, [kf, j_idx]))
        t_v[pl.ds(g * 16, 16)] = tv

    w2 = [[plsc.load_gather(w_v, [jnp.full((16,), _EMB + m, jnp.int32),
                                  jnp.full((16,), j, jnp.int32)])
           for j in range(_OD)] for m in range(_NF)]

    cp_ids.wait()
    cp_num.wait()

    out_cps = []
    for c in range(_CHUNKS):

        @plsc.parallel_loop(c * (_RPC // 16), (c + 1) * (_RPC // 16))
        def _(i):
            rowvec = iota + i * 16
            ids16 = ids_v[pl.ds(i * 16, 16)]
            tbase = ids16 * _OD
            acc = [plsc.load_gather(t_v, [tbase + j]) for j in range(_OD)]
            for m in range(_NF):
                n_m = plsc.load_gather(
                    num_v, [rowvec, jnp.full((16,), m, jnp.int32)])
                for j in range(_OD):
                    acc[j] = acc[j] + n_m * w2[m][j]
            for j in range(_OD):
                plsc.store_scatter(
                    out_v, [rowvec, jnp.full((16,), j, jnp.int32)],
                    jnp.maximum(acc[j], 0.0))

        out_cps.append(pltpu.async_copy(
            out_v.at[pl.ds(c * _RPC, _RPC)],
            out_hbm.at[pl.ds(base + c * _RPC, _RPC)], sem_out))

    for cp in out_cps:
        cp.wait()


@jax.jit
def _run(ids, num, emb, w, b):
    mesh = plsc.VectorSubcoreMesh(core_axis_name="c", subcore_axis_name="s")
    f = pl.kernel(
        _body,
        out_type=jax.ShapeDtypeStruct((_B, _OD), jnp.float32),
        mesh=mesh,
        compiler_params=pltpu.CompilerParams(
            needs_layout_passes=False, use_tc_tiling_on_sc=False),
        scratch_types=[
            pltpu.VMEM((_RPT,), jnp.int32),
            pltpu.VMEM((_RPT, _NF), jnp.float32),
            pltpu.VMEM((_RPT, _OD), jnp.float32),
            pltpu.VMEM((_VOCAB, _EMB), jnp.float32),
            pltpu.VMEM((_EMB + _NF, _OD), jnp.float32),
            pltpu.VMEM((_OD,), jnp.float32),
            pltpu.VMEM((_TPAD,), jnp.float32),
            pltpu.SemaphoreType.DMA,
            pltpu.SemaphoreType.DMA,
            pltpu.SemaphoreType.DMA,
        ],
    )
    return f(ids, num, emb, w, b)


def kernel(code_type_ids_tensor, numerical_props_tensor, emb_table, W, b):
    return _run(code_type_ids_tensor, numerical_props_tensor, emb_table, W, b)


# speedup vs baseline: 1.0018x; 1.0018x over previous
import functools

import jax
import jax.numpy as jnp
from jax import lax
from jax.experimental import pallas as pl
from jax.experimental.pallas import tpu as pltpu
from jax.experimental.pallas import tpu_sc as plsc

_B = 16384
_VOCAB = 5
_EMB = 4
_NF = 3
_OD = 8
_NC = 2
_NS = 16
_NW = _NC * _NS
_RPT = _B // _NW
_TPAD = 48
_CHUNKS = 2
_RPC = _RPT // _CHUNKS


def _body(ids_hbm, num_hbm, emb_hbm, w_hbm, b_hbm, out_hbm,
          ids_v, num_v, out_v, emb_v, w_v, b_v, t_v,
          sem_big, sem_small, sem_out):
    wid = lax.axis_index("s") * _NC + lax.axis_index("c")
    base = wid * _RPT

    cp_ids = pltpu.async_copy(ids_hbm.at[pl.ds(base, _RPT)], ids_v, sem_big)
    cp_num = pltpu.async_copy(num_hbm.at[pl.ds(base, _RPT)], num_v, sem_big)
    cp_emb = pltpu.async_copy(emb_hbm, emb_v, sem_small)
    cp_w = pltpu.async_copy(w_hbm, w_v, sem_small)
    cp_b = pltpu.async_copy(b_hbm, b_v, sem_small)
    cp_emb.wait()
    cp_w.wait()
    cp_b.wait()

    iota = lax.iota(jnp.int32, 16)

    for g in range(_TPAD // 16):
        e = iota + g * 16
        v_idx = jnp.minimum(e >> 3, _VOCAB - 1)
        j_idx = e & 7
        tv = plsc.load_gather(b_v, [j_idx])
        for k in range(_EMB):
            kf = jnp.full((16,), k, jnp.int32)
            tv = tv + (plsc.load_gather(emb_v, [v_idx, kf]) *
                       plsc.load_gather(w_v, [kf, j_idx]))
        t_v[pl.ds(g * 16, 16)] = tv

    w2 = [[plsc.load_gather(w_v, [jnp.full((16,), _EMB + m, jnp.int32),
                                  jnp.full((16,), j, jnp.int32)])
           for j in range(_OD)] for m in range(_NF)]

    cp_ids.wait()
    cp_num.wait()

    out_cps = []
    for c in range(_CHUNKS):

        @plsc.parallel_loop(c * (_RPC // 16), (c + 1) * (_RPC // 16), unroll=2)
        def _(i):
            rowvec = iota + i * 16
            ids16 = ids_v[pl.ds(i * 16, 16)]
            tbase = ids16 * _OD
            acc = [plsc.load_gather(t_v, [tbase + j]) for j in range(_OD)]
            for m in range(_NF):
                n_m = plsc.load_gather(
                    num_v, [rowvec, jnp.full((16,), m, jnp.int32)])
                for j in range(_OD):
                    acc[j] = acc[j] + n_m * w2[m][j]
            for j in range(_OD):
                plsc.store_scatter(
                    out_v, [rowvec, jnp.full((16,), j, jnp.int32)],
                    jnp.maximum(acc[j], 0.0))

        out_cps.append(pltpu.async_copy(
            out_v.at[pl.ds(c * _RPC, _RPC)],
            out_hbm.at[pl.ds(base + c * _RPC, _RPC)], sem_out))

    for cp in out_cps:
        cp.wait()


@jax.jit
def _run(ids, num, emb, w, b):
    mesh = plsc.VectorSubcoreMesh(core_axis_name="c", subcore_axis_name="s")
    f = pl.kernel(
        _body,
        out_type=jax.ShapeDtypeStruct((_B, _OD), jnp.float32),
        mesh=mesh,
        compiler_params=pltpu.CompilerParams(
            needs_layout_passes=False, use_tc_tiling_on_sc=False),
        scratch_types=[
            pltpu.VMEM((_RPT,), jnp.int32),
            pltpu.VMEM((_RPT, _NF), jnp.float32),
            pltpu.VMEM((_RPT, _OD), jnp.float32),
            pltpu.VMEM((_VOCAB, _EMB), jnp.float32),
            pltpu.VMEM((_EMB + _NF, _OD), jnp.float32),
            pltpu.VMEM((_OD,), jnp.float32),
            pltpu.VMEM((_TPAD,), jnp.float32),
            pltpu.SemaphoreType.DMA,
            pltpu.SemaphoreType.DMA,
            pltpu.SemaphoreType.DMA,
        ],
    )
    return f(ids, num, emb, w, b)


def kernel(code_type_ids_tensor, numerical_props_tensor, emb_table, W, b):
    return _run(code_type_ids_tensor, numerical_props_tensor, emb_table, W, b)


# speedup vs baseline: 1.0906x; 1.0886x over previous
import functools

import jax
import jax.numpy as jnp
from jax import lax
from jax.experimental import pallas as pl
from jax.experimental.pallas import tpu as pltpu
from jax.experimental.pallas import tpu_sc as plsc

_B = 16384
_VOCAB = 5
_EMB = 4
_NF = 3
_OD = 8
_NC = 2
_NS = 16
_NW = _NC * _NS
_RPT = _B // _NW
_TPAD = 48
_NROWS = _RPT * _NF // 128


def _body(ids_hbm, num_hbm, emb_hbm, w_hbm, b_hbm, out_hbm,
          ids_v, num_v, out_v, emb_v, w_v, b_v, t_v, sem_big, sem_small):
    wid = lax.axis_index("s") * _NC + lax.axis_index("c")
    base = wid * _RPT

    cp_ids = pltpu.async_copy(ids_hbm.at[pl.ds(base, _RPT)], ids_v, sem_big)
    cp_num = pltpu.async_copy(
        num_hbm.at[pl.ds(wid * _NROWS, _NROWS)], num_v, sem_big)
    cp_emb = pltpu.async_copy(emb_hbm, emb_v, sem_small)
    cp_w = pltpu.async_copy(w_hbm, w_v, sem_small)
    cp_b = pltpu.async_copy(b_hbm, b_v, sem_small)
    cp_emb.wait()
    cp_w.wait()
    cp_b.wait()

    iota = lax.iota(jnp.int32, 16)

    for g in range(_TPAD // 16):
        e = iota + g * 16
        v_idx = jnp.minimum(e >> 3, _VOCAB - 1)
        j_idx = e & 7
        tv = plsc.load_gather(b_v, [j_idx])
        for k in range(_EMB):
            kf = jnp.full((16,), k, jnp.int32)
            tv = tv + (plsc.load_gather(emb_v, [v_idx, kf]) *
                       plsc.load_gather(w_v, [kf, j_idx]))
        t_v[pl.ds(g * 16, 16)] = tv

    w2 = [[plsc.load_gather(w_v, [jnp.full((16,), _EMB + m, jnp.int32),
                                  jnp.full((16,), j, jnp.int32)])
           for j in range(_OD)] for m in range(_NF)]

    cp_ids.wait()
    cp_num.wait()

    @plsc.parallel_loop(0, _RPT // 16)
    def _(i):
        rowvec = iota + i * 16
        ids16 = ids_v[pl.ds(i * 16, 16)]
        tbase = ids16 * _OD
        acc = [plsc.load_gather(t_v, [tbase + j]) for j in range(_OD)]
        rv3 = rowvec * _NF
        for m in range(_NF):
            fm = rv3 + m
            n_m = plsc.load_gather(num_v, [fm >> 7, fm & 127])
            for j in range(_OD):
                acc[j] = acc[j] + n_m * w2[m][j]
        for j in range(_OD):
            plsc.store_scatter(out_v, [rowvec, jnp.full((16,), j, jnp.int32)],
                               jnp.maximum(acc[j], 0.0))

    pltpu.sync_copy(out_v, out_hbm.at[pl.ds(base, _RPT)])


@jax.jit
def _run(ids, num, emb, w, b):
    mesh = plsc.VectorSubcoreMesh(core_axis_name="c", subcore_axis_name="s")
    f = pl.kernel(
        _body,
        out_type=jax.ShapeDtypeStruct((_B, _OD), jnp.float32),
        mesh=mesh,
        compiler_params=pltpu.CompilerParams(
            needs_layout_passes=False, use_tc_tiling_on_sc=False),
        scratch_types=[
            pltpu.VMEM((_RPT,), jnp.int32),
            pltpu.VMEM((_NROWS, 128), jnp.float32),
            pltpu.VMEM((_RPT, _OD), jnp.float32),
            pltpu.VMEM((_VOCAB, _EMB), jnp.float32),
            pltpu.VMEM((_EMB + _NF, _OD), jnp.float32),
            pltpu.VMEM((_OD,), jnp.float32),
            pltpu.VMEM((_TPAD,), jnp.float32),
            pltpu.SemaphoreType.DMA,
            pltpu.SemaphoreType.DMA,
        ],
    )
    return f(ids, num.reshape(_B * _NF // 128, 128), emb, w, b)


def kernel(code_type_ids_tensor, numerical_props_tensor, emb_table, W, b):
    return _run(code_type_ids_tensor, numerical_props_tensor, emb_table, W, b)
